# baseline (speedup 1.0000x reference)
import functools

import jax
import jax.numpy as jnp
from jax import lax
from jax.experimental import pallas as pl
from jax.experimental.pallas import tpu as pltpu
from jax.experimental.pallas import tpu_sc as plsc

_NUM_CORES = 2
_NUM_SUBCORES = 16
_NBUF = 2
_TILESPMEM_WORDS = 131064
_GRAIN = 256


def _chunk_sizes(total: int, nbuf: int):
    cap = (_TILESPMEM_WORDS // nbuf) // _GRAIN * _GRAIN
    nchunk = -(-total // cap)
    while True:
        base = (total // nchunk) // _GRAIN * _GRAIN
        tail = total - base * (nchunk - 1)
        if tail <= cap:
            return [base] * (nchunk - 1) + [tail]
        nchunk += 1


@functools.lru_cache(maxsize=None)
def _make_copy_kernel(nwords: int):
    nworkers = _NUM_CORES * _NUM_SUBCORES
    words_per_w = nwords // nworkers
    sizes = _chunk_sizes(words_per_w, _NBUF)
    offs = [sum(sizes[:i]) for i in range(len(sizes))]
    nchunk = len(sizes)
    nbuf = min(_NBUF, nchunk)
    bufcap = max(sizes)
    mesh = plsc.VectorSubcoreMesh(core_axis_name="c", subcore_axis_name="s")

    @functools.partial(
        pl.kernel,
        mesh=mesh,
        out_type=jax.ShapeDtypeStruct((nwords,), jnp.float32),
        scratch_types=[
            pltpu.VMEM((nbuf, bufcap), jnp.float32),
        ]
        + [pltpu.SemaphoreType.DMA] * (2 * nbuf),
    )
    def k(emb_hbm, out_hbm, buf, *sems):
        rsems = sems[:nbuf]
        wsems = sems[nbuf:]
        wid = lax.axis_index("s") * _NUM_CORES + lax.axis_index("c")
        base = wid * words_per_w

        def read(c):
            b = c % nbuf
            return pltpu.async_copy(
                emb_hbm.at[pl.ds(base + offs[c], sizes[c])],
                buf.at[b, pl.ds(0, sizes[c])], rsems[b])

        def write(c):
            b = c % nbuf
            return pltpu.async_copy(
                buf.at[b, pl.ds(0, sizes[c])],
                out_hbm.at[pl.ds(base + offs[c], sizes[c])], wsems[b])

        reads = {}
        writes = {}
        for c in range(min(nbuf - 1, nchunk)):
            reads[c] = read(c)
        for c in range(nchunk):
            if c + nbuf - 1 < nchunk:
                if c - 1 >= 0:
                    writes.pop(c - 1).wait()
                reads[c + nbuf - 1] = read(c + nbuf - 1)
            reads.pop(c).wait()
            writes[c] = write(c)
        for w in writes.values():
            w.wait()

    return k


def kernel(x, emb_weight):
    seq_len = x.shape[1]
    dim = emb_weight.shape[1]
    flat = emb_weight[:seq_len].reshape(-1)
    out = _make_copy_kernel(flat.shape[0])(flat)
    return out.reshape(seq_len, dim)


# speedup vs baseline: 2.4262x; 2.4262x over previous
import functools

import jax
import jax.numpy as jnp
from jax import lax
from jax.experimental import pallas as pl
from jax.experimental.pallas import tpu as pltpu
from jax.experimental.pallas import tpu_sc as plsc

_NUM_CORES = 2
_NUM_SUBCORES = 16
_NBUF = 2
_TILESPMEM_WORDS = 131064


def _chunk_rows(total_rows: int, dim: int, nbuf: int):
    cap = (_TILESPMEM_WORDS // (nbuf * dim)) // 8 * 8
    nchunk = -(-total_rows // cap)
    while True:
        base = (total_rows // nchunk) // 8 * 8
        tail = total_rows - base * (nchunk - 1)
        if 0 < tail <= cap:
            return [base] * (nchunk - 1) + [tail]
        nchunk += 1


@functools.lru_cache(maxsize=None)
def _make_copy_kernel(seq_len: int, dim: int):
    nworkers = _NUM_CORES * _NUM_SUBCORES
    rows_per_w = seq_len // nworkers
    sizes = _chunk_rows(rows_per_w, dim, _NBUF)
    offs = [sum(sizes[:i]) for i in range(len(sizes))]
    nchunk = len(sizes)
    nbuf = min(_NBUF, nchunk)
    bufcap = max(sizes)
    mesh = plsc.VectorSubcoreMesh(core_axis_name="c", subcore_axis_name="s")

    @functools.partial(
        pl.kernel,
        mesh=mesh,
        out_type=jax.ShapeDtypeStruct((seq_len, dim), jnp.float32),
        scratch_types=[
            pltpu.VMEM((nbuf, bufcap, dim), jnp.float32),
        ]
        + [pltpu.SemaphoreType.DMA] * (2 * nbuf),
    )
    def k(emb_hbm, out_hbm, buf, *sems):
        rsems = sems[:nbuf]
        wsems = sems[nbuf:]
        wid = lax.axis_index("s") * _NUM_CORES + lax.axis_index("c")
        base = wid * rows_per_w

        def read(c):
            b = c % nbuf
            return pltpu.async_copy(
                emb_hbm.at[pl.ds(base + offs[c], sizes[c])],
                buf.at[b, pl.ds(0, sizes[c])], rsems[b])

        def write(c):
            b = c % nbuf
            return pltpu.async_copy(
                buf.at[b, pl.ds(0, sizes[c])],
                out_hbm.at[pl.ds(base + offs[c], sizes[c])], wsems[b])

        reads = {}
        writes = {}
        for c in range(min(nbuf - 1, nchunk)):
            reads[c] = read(c)
        for c in range(nchunk):
            if c + nbuf - 1 < nchunk:
                if c - 1 >= 0:
                    writes.pop(c - 1).wait()
                reads[c + nbuf - 1] = read(c + nbuf - 1)
            reads.pop(c).wait()
            writes[c] = write(c)
        for w in writes.values():
            w.wait()

    return k


def kernel(x, emb_weight):
    seq_len = x.shape[1]
    dim = emb_weight.shape[1]
    return _make_copy_kernel(seq_len, dim)(emb_weight)


# speedup vs baseline: 2.4327x; 1.0027x over previous
import functools

import jax
import jax.numpy as jnp
from jax import lax
from jax.experimental import pallas as pl
from jax.experimental.pallas import tpu as pltpu
from jax.experimental.pallas import tpu_sc as plsc

_NUM_CORES = 2
_NUM_SUBCORES = 16

_TILE_CHUNKS = [24] * 6
_SPMEM_CHUNKS = [32, 32, 24, 24]
_TILE_NBUF = 2
_SPMEM_NBUF = 2


def _ring(read, write, nchunk, nbuf):
    reads = {}
    writes = {}
    for c in range(min(nbuf - 1, nchunk)):
        reads[c] = read(c)
        yield
    for c in range(nchunk):
        if c + nbuf - 1 < nchunk:
            if c - 1 >= 0:
                writes.pop(c - 1).wait()
            reads[c + nbuf - 1] = read(c + nbuf - 1)
        reads.pop(c).wait()
        writes[c] = write(c)
        yield
    for w in writes.values():
        w.wait()
        yield


@functools.lru_cache(maxsize=None)
def _make_copy_kernel(seq_len: int, dim: int):
    nworkers = _NUM_CORES * _NUM_SUBCORES
    rows_per_w = seq_len // nworkers
    t_sizes = list(_TILE_CHUNKS)
    s_sizes = list(_SPMEM_CHUNKS)
    assert sum(t_sizes) + sum(s_sizes) == rows_per_w
    t_offs = [sum(t_sizes[:i]) for i in range(len(t_sizes))]
    s_base0 = sum(t_sizes)
    s_offs = [s_base0 + sum(s_sizes[:i]) for i in range(len(s_sizes))]
    t_nbuf = min(_TILE_NBUF, len(t_sizes))
    s_nbuf = min(_SPMEM_NBUF, len(s_sizes))
    mesh = plsc.VectorSubcoreMesh(core_axis_name="c", subcore_axis_name="s")

    @functools.partial(
        pl.kernel,
        mesh=mesh,
        out_type=jax.ShapeDtypeStruct((seq_len, dim), jnp.float32),
        scratch_types=[
            pltpu.VMEM((t_nbuf, max(t_sizes), dim), jnp.float32),
            pltpu.VMEM_SHARED(
                (_NUM_SUBCORES, s_nbuf, max(s_sizes), dim), jnp.float32),
        ]
        + [pltpu.SemaphoreType.DMA] * (2 * t_nbuf + 2 * s_nbuf),
    )
    def k(emb_hbm, out_hbm, tbuf, sbuf, *sems):
        trs = sems[:t_nbuf]
        tws = sems[t_nbuf:2 * t_nbuf]
        srs = sems[2 * t_nbuf:2 * t_nbuf + s_nbuf]
        sws = sems[2 * t_nbuf + s_nbuf:]
        cid = lax.axis_index("c")
        sid = lax.axis_index("s")
        wid = sid * _NUM_CORES + cid
        base = wid * rows_per_w

        def t_read(c):
            b = c % t_nbuf
            return pltpu.async_copy(
                emb_hbm.at[pl.ds(base + t_offs[c], t_sizes[c])],
                tbuf.at[b, pl.ds(0, t_sizes[c])], trs[b])

        def t_write(c):
            b = c % t_nbuf
            return pltpu.async_copy(
                tbuf.at[b, pl.ds(0, t_sizes[c])],
                out_hbm.at[pl.ds(base + t_offs[c], t_sizes[c])], tws[b])

        def s_read(c):
            b = c % s_nbuf
            return pltpu.async_copy(
                emb_hbm.at[pl.ds(base + s_offs[c], s_sizes[c])],
                sbuf.at[sid, b, pl.ds(0, s_sizes[c])], srs[b])

        def s_write(c):
            b = c % s_nbuf
            return pltpu.async_copy(
                sbuf.at[sid, b, pl.ds(0, s_sizes[c])],
                out_hbm.at[pl.ds(base + s_offs[c], s_sizes[c])], sws[b])

        rings = [
            _ring(t_read, t_write, len(t_sizes), t_nbuf),
            _ring(s_read, s_write, len(s_sizes), s_nbuf),
        ]
        while rings:
            nxt = []
            for g in rings:
                try:
                    next(g)
                    nxt.append(g)
                except StopIteration:
                    pass
            rings = nxt

    return k


def kernel(x, emb_weight):
    seq_len = x.shape[1]
    dim = emb_weight.shape[1]
    return _make_copy_kernel(seq_len, dim)(emb_weight)


# speedup vs baseline: 2.5101x; 1.0318x over previous
import functools

import jax
import jax.numpy as jnp
from jax import lax
from jax.experimental import pallas as pl
from jax.experimental.pallas import tpu as pltpu
from jax.experimental.pallas import tpu_sc as plsc

_NUM_CORES = 2
_NUM_SUBCORES = 16
_NBUF = 3
_CHUNKS_256 = [40, 40, 40, 40, 40, 40, 16]


def _chunk_rows(total_rows: int, dim: int, nbuf: int):
    if total_rows == 256 and dim == 1024:
        return list(_CHUNKS_256)
    cap = max(8, (131064 // (nbuf * dim)) // 8 * 8)
    nchunk = -(-total_rows // cap)
    while True:
        base = (total_rows // nchunk) // 8 * 8
        tail = total_rows - base * (nchunk - 1)
        if 0 < tail <= cap:
            return [base] * (nchunk - 1) + [tail]
        nchunk += 1


@functools.lru_cache(maxsize=None)
def _make_copy_kernel(seq_len: int, dim: int):
    nworkers = _NUM_CORES * _NUM_SUBCORES
    rows_per_w = seq_len // nworkers
    sizes = _chunk_rows(rows_per_w, dim, _NBUF)
    offs = [sum(sizes[:i]) for i in range(len(sizes))]
    nchunk = len(sizes)
    nbuf = min(_NBUF, nchunk)
    bufcap = max(sizes)
    mesh = plsc.VectorSubcoreMesh(core_axis_name="c", subcore_axis_name="s")

    @functools.partial(
        pl.kernel,
        mesh=mesh,
        out_type=jax.ShapeDtypeStruct((seq_len, dim), jnp.float32),
        scratch_types=[
            pltpu.VMEM((nbuf, bufcap, dim), jnp.float32),
        ]
        + [pltpu.SemaphoreType.DMA] * (2 * nbuf),
    )
    def k(emb_hbm, out_hbm, buf, *sems):
        rsems = sems[:nbuf]
        wsems = sems[nbuf:]
        wid = lax.axis_index("s") * _NUM_CORES + lax.axis_index("c")
        base = wid * rows_per_w

        def read(c):
            b = c % nbuf
            return pltpu.async_copy(
                emb_hbm.at[pl.ds(base + offs[c], sizes[c])],
                buf.at[b, pl.ds(0, sizes[c])], rsems[b])

        def write(c):
            b = c % nbuf
            return pltpu.async_copy(
                buf.at[b, pl.ds(0, sizes[c])],
                out_hbm.at[pl.ds(base + offs[c], sizes[c])], wsems[b])

        reads = {}
        writes = {}
        for c in range(min(nbuf - 1, nchunk)):
            reads[c] = read(c)
        for c in range(nchunk):
            if c + nbuf - 1 < nchunk:
                if c - 1 >= 0:
                    writes.pop(c - 1).wait()
                reads[c + nbuf - 1] = read(c + nbuf - 1)
            reads.pop(c).wait()
            writes[c] = write(c)
        for w in writes.values():
            w.wait()

    return k


def kernel(x, emb_weight):
    seq_len = x.shape[1]
    dim = emb_weight.shape[1]
    return _make_copy_kernel(seq_len, dim)(emb_weight)
